# baseline (device time: 700792 ns/iter reference)
import jax
import jax.numpy as jnp
from jax import lax
from jax.experimental import pallas as pl
from jax.experimental.pallas import tpu as pltpu

N_DEV = 4
TC = 256
G = 16


def kernel(x, A, B, C):
    Bb, S, D = x.shape
    N = B.shape[-1]

    def body(x_ref, a_ref, b_ref, c_ref, y_ref,
             hbuf, send_buf, recv_buf, send_sem, recv_sem):
        my = lax.axis_index("i")
        right = (my + 1) % N_DEV

        da = jnp.exp(a_ref[...]).T[None]

        def scan_block(n_groups, h_init):
            def group(gi, h):
                t0 = gi * G
                xg = x_ref[:, pl.ds(t0, G), :]
                bg = jnp.transpose(b_ref[:, pl.ds(t0, G), :], (0, 2, 1))
                cg = jnp.transpose(c_ref[:, pl.ds(t0, G), :], (0, 2, 1))
                ug = xg[:, None, :, :] * bg[:, :, :, None]
                for k in range(G):
                    h = h * da + ug[:, :, k, :]
                    hbuf[:, :, k, :] = h
                p = hbuf[...] * cg[:, :, :, None]
                y_ref[:, pl.ds(t0, G), :] = jnp.sum(p, axis=1)
                return h

            return lax.fori_loop(0, n_groups, group, h_init)

        h_final = scan_block(S // G, jnp.zeros((Bb, N, D), jnp.float32))
        send_buf[...] = h_final

        cp = pltpu.make_async_remote_copy(
            src_ref=send_buf,
            dst_ref=recv_buf,
            send_sem=send_sem,
            recv_sem=recv_sem,
            device_id=(right,),
            device_id_type=pl.DeviceIdType.MESH,
        )

        @pl.when(my > 0)
        def _():
            cp.wait_recv()
            pow_s = jnp.exp(a_ref[...] * float(S)).T[None]
            send_buf[...] = pow_s * recv_buf[...] + send_buf[...]

        @pl.when(my < N_DEV - 1)
        def _():
            cp.start()

        @pl.when(my > 0)
        def _():
            scan_block(TC // G, recv_buf[...])

        @pl.when(my < N_DEV - 1)
        def _():
            cp.wait_send()

    return pl.pallas_call(
        body,
        out_shape=jax.ShapeDtypeStruct((Bb, S, D), jnp.float32),
        in_specs=[pl.BlockSpec(memory_space=pltpu.VMEM)] * 4,
        out_specs=pl.BlockSpec(memory_space=pltpu.VMEM),
        scratch_shapes=[
            pltpu.VMEM((Bb, N, G, D), jnp.float32),
            pltpu.VMEM((Bb, N, D), jnp.float32),
            pltpu.VMEM((Bb, N, D), jnp.float32),
            pltpu.SemaphoreType.DMA,
            pltpu.SemaphoreType.DMA,
        ],
        compiler_params=pltpu.CompilerParams(
            vmem_limit_bytes=100 * 1024 * 1024,
        ),
    )(x, A, B, C)


# device time: 260772 ns/iter; 2.6874x vs baseline; 2.6874x over previous
import jax
import jax.numpy as jnp
from jax import lax
from jax.experimental import pallas as pl
from jax.experimental.pallas import tpu as pltpu

N_DEV = 4
TC = 256
G = 8


def kernel(x, A, B, C):
    Bb, S, D = x.shape
    N = B.shape[-1]

    def body(x_ref, a_ref, b_ref, c_ref, y_ref,
             send_buf, recv_buf, send_sem, recv_sem):
        my = lax.axis_index("i")
        right = (my + 1) % N_DEV

        da = jnp.exp(a_ref[...]).T[None]

        def scan_block(n_groups, h_init):
            def group(gi, h):
                t0 = gi * G
                xg = x_ref[:, pl.ds(t0, G), :]
                bg = jnp.transpose(b_ref[:, pl.ds(t0, G), :], (0, 2, 1))
                cg = jnp.transpose(c_ref[:, pl.ds(t0, G), :], (0, 2, 1))
                ys = []
                for k in range(G):
                    h = h * da + xg[:, k:k + 1, :] * bg[:, :, k:k + 1]
                    ys.append(jnp.sum(h * cg[:, :, k:k + 1], axis=1, keepdims=True))
                y_ref[:, pl.ds(t0, G), :] = jnp.concatenate(ys, axis=1)
                return h

            return lax.fori_loop(0, n_groups, group, h_init)

        h_final = scan_block(S // G, jnp.zeros((Bb, N, D), jnp.float32))
        send_buf[...] = h_final

        cp = pltpu.make_async_remote_copy(
            src_ref=send_buf,
            dst_ref=recv_buf,
            send_sem=send_sem,
            recv_sem=recv_sem,
            device_id=(right,),
            device_id_type=pl.DeviceIdType.MESH,
        )

        @pl.when(my > 0)
        def _():
            cp.wait_recv()
            pow_s = jnp.exp(a_ref[...] * float(S)).T[None]
            send_buf[...] = pow_s * recv_buf[...] + send_buf[...]

        @pl.when(my < N_DEV - 1)
        def _():
            cp.start()

        @pl.when(my > 0)
        def _():
            scan_block(TC // G, recv_buf[...])

        @pl.when(my < N_DEV - 1)
        def _():
            cp.wait_send()

    return pl.pallas_call(
        body,
        out_shape=jax.ShapeDtypeStruct((Bb, S, D), jnp.float32),
        in_specs=[pl.BlockSpec(memory_space=pltpu.VMEM)] * 4,
        out_specs=pl.BlockSpec(memory_space=pltpu.VMEM),
        scratch_shapes=[
            pltpu.VMEM((Bb, N, D), jnp.float32),
            pltpu.VMEM((Bb, N, D), jnp.float32),
            pltpu.SemaphoreType.DMA,
            pltpu.SemaphoreType.DMA,
        ],
        compiler_params=pltpu.CompilerParams(
            vmem_limit_bytes=100 * 1024 * 1024,
        ),
    )(x, A, B, C)
